# baseline (speedup 1.0000x reference)
import functools

import jax
import jax.numpy as jnp
from jax.experimental import pallas as pl
from jax.experimental.pallas import tpu as pltpu

_CHUNK = 512


def _scorer_body(nq, hw, nb, b_neigh, fv_ref, mb_ref, pix_ref, img_ref,
                 fvb_ref, mbb_ref, mbn_ref, acc_ref, d2_ref):
    batch = nq // hw
    d = fv_ref.shape[1]
    fv = fv_ref[...]
    ones_row = jnp.ones((1, d), jnp.float32)

    fvb_ref[:, :d] = (fv * (-2.0)).astype(jnp.bfloat16)
    fvb_ref[:, d:] = jnp.concatenate(
        [jnp.ones((nq, 2), jnp.bfloat16), jnp.zeros((nq, 6), jnp.bfloat16)],
        axis=1)
    for c in range(8):
        mc = mb_ref[pl.ds(c * (nb // 8), nb // 8), :]
        mbb_ref[pl.ds(c * (nb // 8), nb // 8), :d] = mc.astype(jnp.bfloat16)
        mbn_ref[0:1, pl.ds(c * (nb // 8), nb // 8)] = jax.lax.dot_general(
            ones_row, mc * mc, (((1,), (1,)), ((), ())),
            preferred_element_type=jnp.float32)
    hi_row = mbn_ref[...].astype(jnp.bfloat16)
    lo_row = (mbn_ref[...] - hi_row.astype(jnp.float32)).astype(jnp.bfloat16)
    mbb_ref[:, d:d + 1] = jnp.reshape(hi_row, (nb, 1))
    mbb_ref[:, d + 1:d + 2] = jnp.reshape(lo_row, (nb, 1))
    mbb_ref[:, d + 2:] = jnp.zeros((nb, 6), jnp.bfloat16)

    acc_ref[...] = jnp.full(acc_ref.shape, jnp.inf, jnp.float32)
    fvb = fvb_ref[...]
    _UNROLL = 2
    n_outer = nb // (_CHUNK * _UNROLL)

    def min_step(i, _):
        base = i * (_CHUNK * _UNROLL)
        ts = []
        for u in range(_UNROLL):
            off = base + u * _CHUNK
            pa = jax.lax.dot_general(fvb, mbb_ref[pl.ds(off, _CHUNK), :],
                                     (((1,), (1,)), ((), ())),
                                     preferred_element_type=jnp.float32)
            t = None
            for j in range(_CHUNK // d):
                blk = pa[:, j * d:(j + 1) * d]
                t = blk if t is None else jnp.minimum(t, blk)
            ts.append(t)
        while len(ts) > 1:
            ts = [jnp.minimum(ts[k], ts[k + 1]) for k in range(0, len(ts), 2)]
        acc_ref[...] = jnp.minimum(acc_ref[...], ts[0])
        return 0

    jax.lax.fori_loop(0, n_outer, min_step, 0)

    fvn = jnp.sum(fv * fv, axis=1, keepdims=True)
    rowmin = jnp.min(acc_ref[...], axis=1, keepdims=True)
    pix = jnp.sqrt(rowmin + fvn)
    pix_ref[...] = pix

    sels = []
    for b in range(batch):
        seg = pix[b * hw:(b + 1) * hw, :]
        m = jnp.max(seg)
        io = jax.lax.broadcasted_iota(jnp.int32, (hw, 1), 0)
        first = jnp.min(jnp.where(seg == m, io, hw))
        sels.append(fv_ref[pl.ds(b * hw + first, 1), :])
    sel = jnp.concatenate(sels, axis=0)
    selm2 = sel * (-2.0)
    seln = jnp.sum(sel * sel, axis=1, keepdims=True)

    prod2 = jax.lax.dot_general(selm2, mb_ref[...],
                                (((1,), (1,)), ((), ())),
                                preferred_element_type=jnp.float32)
    d2_ref[...] = prod2 + mbn_ref[...] + seln

    io2 = jax.lax.broadcasted_iota(jnp.int32, (batch, nb), 1)
    vals = []
    for _ in range(b_neigh):
        d = d2_ref[...]
        mk = jnp.min(d, axis=1, keepdims=True)
        vals.append(mk)
        fk = jnp.min(jnp.where(d == mk, io2, jnp.int32(2 ** 30)),
                     axis=1, keepdims=True)
        d2_ref[...] = jnp.where(io2 == fk, jnp.inf, d)

    sd = jnp.sqrt(jnp.concatenate(vals, axis=1))
    mx = jnp.max(sd, axis=1, keepdims=True)
    e = jnp.exp(sd - mx)
    p0 = e[:, 0:1] / jnp.sum(e, axis=1, keepdims=True)
    img_ref[...] = sd[:, 0:1] * (1.0 - p0)


def kernel(feature_batch, mb):
    batch, height, width, channels = feature_batch.shape
    nq = batch * height * width
    hw = height * width
    nb = mb.shape[0]
    b_neigh = 9
    fv = jnp.reshape(feature_batch, (nq, channels))

    body = functools.partial(_scorer_body, nq, hw, nb, b_neigh)
    pix, img = pl.pallas_call(
        body,
        out_shape=(
            jax.ShapeDtypeStruct((nq, 1), jnp.float32),
            jax.ShapeDtypeStruct((batch, 1), jnp.float32),
        ),
        scratch_shapes=[
            pltpu.VMEM((nq, channels + 8), jnp.bfloat16),
            pltpu.VMEM((nb, channels + 8), jnp.bfloat16),
            pltpu.VMEM((1, nb), jnp.float32),
            pltpu.VMEM((nq, channels), jnp.float32),
            pltpu.VMEM((batch, nb), jnp.float32),
        ],
    )(fv, mb)
    return (jnp.reshape(pix, (batch, 1, height, width)),
            jnp.reshape(img, (batch,)))


# speedup vs baseline: 1.1008x; 1.1008x over previous
import functools

import jax
import jax.numpy as jnp
from jax.experimental import pallas as pl
from jax.experimental.pallas import tpu as pltpu

_CHUNK = 512


def _scorer_body(nq, hw, nb, b_neigh, fv_ref, mb_ref, pix_ref, img_ref,
                 fvb_ref, mbb_ref, mbn_ref, acc_ref, d2_ref):
    batch = nq // hw
    d = fv_ref.shape[1]
    fv = fv_ref[...]
    ones_row = jnp.ones((1, d), jnp.float32)

    fvb_ref[...] = (fv * (-2.0)).astype(jnp.bfloat16)
    for c in range(8):
        mc = mb_ref[pl.ds(c * (nb // 8), nb // 8), :]
        mbb_ref[pl.ds(c * (nb // 8), nb // 8), :] = mc.astype(jnp.bfloat16)
        mbn_ref[0:1, pl.ds(c * (nb // 8), nb // 8)] = jax.lax.dot_general(
            ones_row, mc * mc, (((1,), (1,)), ((), ())),
            preferred_element_type=jnp.float32)

    acc_ref[...] = jnp.full(acc_ref.shape, jnp.inf, jnp.float32)
    fvb = fvb_ref[...]
    _UNROLL = 4
    n_outer = nb // (_CHUNK * _UNROLL)

    def min_step(i, _):
        base = i * (_CHUNK * _UNROLL)
        for u in range(_UNROLL):
            off = base + u * _CHUNK
            pa = jax.lax.dot_general(fvb, mbb_ref[pl.ds(off, _CHUNK), :],
                                     (((1,), (1,)), ((), ())),
                                     preferred_element_type=jnp.float32)
            t = None
            for j in range(_CHUNK // d):
                nrow = mbn_ref[0:1, pl.ds(off + j * d, d)]
                blk = pa[:, j * d:(j + 1) * d] + nrow
                t = blk if t is None else jnp.minimum(t, blk)
            acc_ref[...] = jnp.minimum(acc_ref[...], t)
        return 0

    jax.lax.fori_loop(0, n_outer, min_step, 0)

    fvn = jnp.sum(fv * fv, axis=1, keepdims=True)
    rowmin = jnp.min(acc_ref[...], axis=1, keepdims=True)
    pix = jnp.sqrt(rowmin + fvn)
    pix_ref[...] = pix

    sels = []
    for b in range(batch):
        seg = pix[b * hw:(b + 1) * hw, :]
        m = jnp.max(seg)
        io = jax.lax.broadcasted_iota(jnp.int32, (hw, 1), 0)
        first = jnp.min(jnp.where(seg == m, io, hw))
        sels.append(fv_ref[pl.ds(b * hw + first, 1), :])
    sel = jnp.concatenate(sels, axis=0)
    selm2 = sel * (-2.0)
    seln = jnp.sum(sel * sel, axis=1, keepdims=True)

    nb2 = nb // 2
    for h in range(2):
        pr = jax.lax.dot_general(selm2, mb_ref[pl.ds(h * nb2, nb2), :],
                                 (((1,), (1,)), ((), ())),
                                 preferred_element_type=jnp.float32)
        d2_ref[h * batch:(h + 1) * batch, :] = (
            pr + mbn_ref[0:1, pl.ds(h * nb2, nb2)] + seln)

    rowh = jax.lax.broadcasted_iota(jnp.int32, (2 * batch, nb2), 0) // batch
    gidx = jax.lax.broadcasted_iota(jnp.int32, (2 * batch, nb2), 1) + rowh * nb2
    vals = []
    for _ in range(b_neigh):
        dcur = d2_ref[...]
        mh = jnp.min(dcur, axis=1, keepdims=True)
        mq = jnp.minimum(mh[0:batch, :], mh[batch:, :])
        vals.append(mq)
        mfull = jnp.concatenate([mq, mq], axis=0)
        cand = jnp.where(dcur == mfull, gidx, jnp.int32(2 ** 30))
        fh = jnp.min(cand, axis=1, keepdims=True)
        fq = jnp.minimum(fh[0:batch, :], fh[batch:, :])
        ffull = jnp.concatenate([fq, fq], axis=0)
        d2_ref[...] = jnp.where(gidx == ffull, jnp.inf, dcur)

    sd = jnp.sqrt(jnp.concatenate(vals, axis=1))
    mx = jnp.max(sd, axis=1, keepdims=True)
    e = jnp.exp(sd - mx)
    p0 = e[:, 0:1] / jnp.sum(e, axis=1, keepdims=True)
    img_ref[...] = sd[:, 0:1] * (1.0 - p0)


def kernel(feature_batch, mb):
    batch, height, width, channels = feature_batch.shape
    nq = batch * height * width
    hw = height * width
    nb = mb.shape[0]
    b_neigh = 9
    fv = jnp.reshape(feature_batch, (nq, channels))

    body = functools.partial(_scorer_body, nq, hw, nb, b_neigh)
    pix, img = pl.pallas_call(
        body,
        out_shape=(
            jax.ShapeDtypeStruct((nq, 1), jnp.float32),
            jax.ShapeDtypeStruct((batch, 1), jnp.float32),
        ),
        scratch_shapes=[
            pltpu.VMEM((nq, channels), jnp.bfloat16),
            pltpu.VMEM((nb, channels), jnp.bfloat16),
            pltpu.VMEM((1, nb), jnp.float32),
            pltpu.VMEM((nq, channels), jnp.float32),
            pltpu.VMEM((2 * batch, nb // 2), jnp.float32),
        ],
    )(fv, mb)
    return (jnp.reshape(pix, (batch, 1, height, width)),
            jnp.reshape(img, (batch,)))
